# Initial kernel scaffold; baseline (speedup 1.0000x reference)
import jax
import jax.numpy as jnp
from jax.experimental import pallas as pl


def kernel(x, edge_index, batch, W1_l, b1_l, W1_r, W2_l, b2_l, W2_r):
    raise NotImplementedError("write your pallas kernel here")


# speedup vs baseline: 21.5179x; 21.5179x over previous
import functools

import jax
import jax.numpy as jnp
from jax import lax
from jax.experimental import pallas as pl
from jax.experimental.pallas import tpu as pltpu
from jax.experimental.pallas import tpu_sc as plsc

N = 10000
E = 320000
D = 128
H = 16
O = 16
G = 128

NC = 2
NS = 16
NW = NC * NS
EP = E // NW
CH = 2000
NCH = EP // CH
NZ = 632
WB = 632
SH_N = NS * NZ

BN = 2000
NB = N // BN


def _sc_scatter_body(with_deg, *refs):
    if with_deg:
        (p_hbm, src_hbm, dst_hbm, agg0, agg1, deg0, deg1,
         idx_v, dst_v, rows_v, ones_v, zb_v, agg_sh, deg_sh, sem) = refs
    else:
        (p_hbm, src_hbm, dst_hbm, agg0, agg1,
         idx_v, dst_v, rows_v, zb_v, agg_sh, sem) = refs

    c = lax.axis_index("c")
    s = lax.axis_index("s")
    wid = s * NC + c
    base = wid * EP

    def _zrow(i, _):
        zb_v[i, :] = jnp.zeros((16,), jnp.float32)
        return 0
    lax.fori_loop(0, NZ, _zrow, 0)
    if with_deg:
        def _orow(i, _):
            ones_v[i, :] = jnp.ones((16,), jnp.float32)
            return 0
        lax.fori_loop(0, CH, _orow, 0)

    pltpu.sync_copy(zb_v, agg_sh.at[pl.ds(s * NZ, NZ)])
    if with_deg:
        pltpu.sync_copy(zb_v, deg_sh.at[pl.ds(s * NZ, NZ)])
    plsc.subcore_barrier()

    def _chunk(i, _):
        off = base + i * CH
        pltpu.sync_copy(src_hbm.at[pl.ds(off, CH)], idx_v)
        pltpu.sync_copy(dst_hbm.at[pl.ds(off, CH)], dst_v)
        pltpu.async_copy(p_hbm.at[idx_v], rows_v, sem).wait()
        pltpu.sync_copy(rows_v, agg_sh.at[dst_v], add=True)
        if with_deg:
            pltpu.sync_copy(ones_v, deg_sh.at[dst_v], add=True)
        return 0
    lax.fori_loop(0, NCH, _chunk, 0)
    plsc.subcore_barrier()

    rb = s * WB

    @pl.when(c == 0)
    def _():
        pltpu.sync_copy(agg_sh.at[pl.ds(rb, WB)], agg0.at[pl.ds(rb, WB)])
        if with_deg:
            pltpu.sync_copy(deg_sh.at[pl.ds(rb, WB)], deg0.at[pl.ds(rb, WB)])

    @pl.when(c == 1)
    def _():
        pltpu.sync_copy(agg_sh.at[pl.ds(rb, WB)], agg1.at[pl.ds(rb, WB)])
        if with_deg:
            pltpu.sync_copy(deg_sh.at[pl.ds(rb, WB)], deg1.at[pl.ds(rb, WB)])


def _make_sc_scatter(with_deg):
    n_out = 4 if with_deg else 2
    scratch = [
        pltpu.VMEM((CH,), jnp.int32),
        pltpu.VMEM((CH,), jnp.int32),
        pltpu.VMEM((CH, 16), jnp.float32),
    ]
    if with_deg:
        scratch.append(pltpu.VMEM((CH, 16), jnp.float32))
    scratch.append(pltpu.VMEM((NZ, 16), jnp.float32))
    scratch.append(pltpu.VMEM_SHARED((SH_N, 16), jnp.float32))
    if with_deg:
        scratch.append(pltpu.VMEM_SHARED((SH_N, 16), jnp.float32))
    scratch.append(pltpu.SemaphoreType.DMA)

    mesh = plsc.VectorSubcoreMesh(core_axis_name="c", subcore_axis_name="s")
    return functools.partial(
        pl.kernel,
        mesh=mesh,
        out_type=[jax.ShapeDtypeStruct((SH_N, 16), jnp.float32)] * n_out,
        scratch_types=scratch,
        compiler_params=pltpu.CompilerParams(use_tc_tiling_on_sc=False),
    )(functools.partial(_sc_scatter_body, with_deg))


def _proj1_body(x_ref, wl_ref, wr_ref, p_ref, r_ref):
    xb = x_ref[...]
    dn = (((1,), (1,)), ((), ()))
    p_ref[...] = lax.dot_general(xb, wl_ref[...], dn,
                                 preferred_element_type=jnp.float32)
    r_ref[...] = lax.dot_general(xb, wr_ref[...], dn,
                                 preferred_element_type=jnp.float32)


def _mid_body(a0_ref, a1_ref, d0_ref, d1_ref, r1_ref, b1_ref,
              w2l_ref, w2r_ref, b2_ref, p2_ref, r2b_ref):
    agg = a0_ref[...] + a1_ref[...]
    deg = jnp.maximum(d0_ref[...] + d1_ref[...], 1.0)
    h = jnp.maximum(agg / deg + b1_ref[...] + r1_ref[...], 0.0)
    dn = (((1,), (1,)), ((), ()))
    p2_ref[...] = lax.dot_general(h, w2l_ref[...], dn,
                                  preferred_element_type=jnp.float32)
    r2b_ref[...] = lax.dot_general(h, w2r_ref[...], dn,
                                   preferred_element_type=jnp.float32) + b2_ref[...]


def _final_body(a0_ref, a1_ref, d0_ref, d1_ref, r2b_ref, batch_ref,
                out_ref, sum_acc, cnt_acc):
    i = pl.program_id(0)

    @pl.when(i == 0)
    def _():
        sum_acc[...] = jnp.zeros_like(sum_acc)
        cnt_acc[...] = jnp.zeros_like(cnt_acc)

    agg = a0_ref[...] + a1_ref[...]
    deg = jnp.maximum(d0_ref[...] + d1_ref[...], 1.0)
    node = agg / deg + r2b_ref[...]

    bvec = batch_ref[0]
    ohT = (lax.broadcasted_iota(jnp.int32, (G, BN), 0)
           == jnp.broadcast_to(bvec, (G, BN))).astype(jnp.float32)
    dn = (((1,), (0,)), ((), ()))
    sum_acc[...] += lax.dot_general(ohT, node, dn,
                                    preferred_element_type=jnp.float32)
    cnt_acc[...] += jnp.sum(ohT, axis=1, keepdims=True)

    @pl.when(i == pl.num_programs(0) - 1)
    def _():
        out_ref[...] = sum_acc[...] / jnp.maximum(cnt_acc[...], 1.0)


def _row_spec(w):
    return pl.BlockSpec((BN, w), lambda i: (i, 0))


def _full_spec(shape):
    return pl.BlockSpec(shape, lambda i: tuple(0 for _ in shape))


def _proj1(x, W1_l, W1_r):
    return pl.pallas_call(
        _proj1_body,
        grid=(NB,),
        in_specs=[_row_spec(D), _full_spec((H, D)), _full_spec((H, D))],
        out_specs=[_row_spec(H), _row_spec(H)],
        out_shape=[jax.ShapeDtypeStruct((N, H), jnp.float32)] * 2,
    )(x, W1_l, W1_r)


def _mid(a0, a1, d0, d1, R1, b1, W2_l, W2_r, b2):
    return pl.pallas_call(
        _mid_body,
        grid=(NB,),
        in_specs=[_row_spec(16), _row_spec(16), _row_spec(16), _row_spec(16),
                  _row_spec(H), _full_spec((1, H)),
                  _full_spec((O, H)), _full_spec((O, H)), _full_spec((1, O))],
        out_specs=[_row_spec(O), _row_spec(O)],
        out_shape=[jax.ShapeDtypeStruct((N, O), jnp.float32)] * 2,
    )(a0, a1, d0, d1, R1, b1, W2_l, W2_r, b2)


def _final(a0, a1, d0, d1, R2b, batch3):
    return pl.pallas_call(
        _final_body,
        grid=(NB,),
        in_specs=[_row_spec(16), _row_spec(16), _row_spec(16), _row_spec(16),
                  _row_spec(O),
                  pl.BlockSpec((1, 1, BN), lambda i: (i, 0, 0))],
        out_specs=pl.BlockSpec((G, O), lambda i: (0, 0)),
        out_shape=jax.ShapeDtypeStruct((G, O), jnp.float32),
        scratch_shapes=[pltpu.VMEM((G, O), jnp.float32),
                        pltpu.VMEM((G, 1), jnp.float32)],
    )(a0, a1, d0, d1, R2b, batch3)


_sc_pass1 = _make_sc_scatter(True)
_sc_pass2 = _make_sc_scatter(False)


def kernel(x, edge_index, batch, W1_l, b1_l, W1_r, W2_l, b2_l, W2_r):
    src = edge_index[0].astype(jnp.int32)
    dst = edge_index[1].astype(jnp.int32)
    b1 = b1_l.reshape(1, H).astype(jnp.float32)
    b2 = b2_l.reshape(1, O).astype(jnp.float32)
    batch3 = batch.astype(jnp.int32).reshape(NB, 1, BN)
    x = x.astype(jnp.float32)

    P1, R1 = _proj1(x, W1_l, W1_r)
    a0, a1, d0, d1 = _sc_pass1(P1, src, dst)
    P2, R2b = _mid(a0, a1, d0, d1, R1, b1, W2_l, W2_r, b2)
    g0, g1 = _sc_pass2(P2, src, dst)
    return _final(g0, g1, d0, d1, R2b, batch3)


# speedup vs baseline: 25.3772x; 1.1794x over previous
import functools

import jax
import jax.numpy as jnp
from jax import lax
from jax.experimental import pallas as pl
from jax.experimental.pallas import tpu as pltpu
from jax.experimental.pallas import tpu_sc as plsc

N = 10000
E = 320000
D = 128
H = 16
O = 16
G = 128

NC = 2
NS = 16
NW = NC * NS
EP = E // NW
CH = 1000
NCH = EP // CH
NZ = 632
WB = 632
SH_N = NS * NZ

BN = 2000
NB = N // BN


def _sc_scatter_body(with_deg, *refs):
    if with_deg:
        (p_hbm, src_hbm, dst_hbm, agg0, agg1, deg0, deg1,
         idx_all, dst_all, rows0, rows1, ones_v, agg_sh, deg_sh,
         isem, gsem, asem, dsem) = refs
    else:
        (p_hbm, src_hbm, dst_hbm, agg0, agg1,
         idx_all, dst_all, rows0, rows1, agg_sh,
         isem, gsem, asem) = refs

    c = lax.axis_index("c")
    s = lax.axis_index("s")
    wid = s * NC + c
    base = wid * EP

    idesc = []
    for i in range(NCH):
        off = base + i * CH
        idesc.append(pltpu.async_copy(
            src_hbm.at[pl.ds(off, CH)], idx_all.at[i], isem))
        idesc.append(pltpu.async_copy(
            dst_hbm.at[pl.ds(off, CH)], dst_all.at[i], isem))

    def _zrow(i, _):
        rows0[i, :] = jnp.zeros((16,), jnp.float32)
        return 0
    lax.fori_loop(0, NZ, _zrow, 0)
    if with_deg:
        def _orow(i, _):
            ones_v[i, :] = jnp.ones((16,), jnp.float32)
            return 0
        lax.fori_loop(0, CH, _orow, 0)

    pltpu.sync_copy(rows0.at[pl.ds(0, NZ)], agg_sh.at[pl.ds(s * NZ, NZ)])
    if with_deg:
        pltpu.sync_copy(rows0.at[pl.ds(0, NZ)], deg_sh.at[pl.ds(s * NZ, NZ)])
    plsc.subcore_barrier()
    for d in idesc:
        d.wait()

    rows = (rows0, rows1)
    g = [None] * NCH
    a = [None] * NCH
    dd = []
    g[0] = pltpu.async_copy(p_hbm.at[idx_all.at[0]], rows[0], gsem)
    for i in range(NCH):
        if i + 1 < NCH:
            if i >= 1:
                a[i - 1].wait()
            g[i + 1] = pltpu.async_copy(
                p_hbm.at[idx_all.at[i + 1]], rows[(i + 1) % 2], gsem)
        g[i].wait()
        a[i] = pltpu.async_copy(
            rows[i % 2], agg_sh.at[dst_all.at[i]], asem, add=True)
        if with_deg:
            dd.append(pltpu.async_copy(
                ones_v, deg_sh.at[dst_all.at[i]], dsem, add=True))
    for i in range(max(0, NCH - 2), NCH):
        a[i].wait()
    for d in dd:
        d.wait()
    plsc.subcore_barrier()

    rb = s * WB

    @pl.when(c == 0)
    def _():
        pltpu.sync_copy(agg_sh.at[pl.ds(rb, WB)], agg0.at[pl.ds(rb, WB)])
        if with_deg:
            pltpu.sync_copy(deg_sh.at[pl.ds(rb, WB)], deg0.at[pl.ds(rb, WB)])

    @pl.when(c == 1)
    def _():
        pltpu.sync_copy(agg_sh.at[pl.ds(rb, WB)], agg1.at[pl.ds(rb, WB)])
        if with_deg:
            pltpu.sync_copy(deg_sh.at[pl.ds(rb, WB)], deg1.at[pl.ds(rb, WB)])


def _make_sc_scatter(with_deg):
    n_out = 4 if with_deg else 2
    scratch = [
        pltpu.VMEM((NCH, CH), jnp.int32),
        pltpu.VMEM((NCH, CH), jnp.int32),
        pltpu.VMEM((CH, 16), jnp.float32),
        pltpu.VMEM((CH, 16), jnp.float32),
    ]
    if with_deg:
        scratch.append(pltpu.VMEM((CH, 16), jnp.float32))
    scratch.append(pltpu.VMEM_SHARED((SH_N, 16), jnp.float32))
    if with_deg:
        scratch.append(pltpu.VMEM_SHARED((SH_N, 16), jnp.float32))
    scratch.extend([pltpu.SemaphoreType.DMA] * (4 if with_deg else 3))

    mesh = plsc.VectorSubcoreMesh(core_axis_name="c", subcore_axis_name="s")
    return functools.partial(
        pl.kernel,
        mesh=mesh,
        out_type=[jax.ShapeDtypeStruct((SH_N, 16), jnp.float32)] * n_out,
        scratch_types=scratch,
        compiler_params=pltpu.CompilerParams(use_tc_tiling_on_sc=False),
    )(functools.partial(_sc_scatter_body, with_deg))


def _proj1_body(x_ref, wl_ref, wr_ref, p_ref, r_ref):
    xb = x_ref[...]
    dn = (((1,), (1,)), ((), ()))
    p_ref[...] = lax.dot_general(xb, wl_ref[...], dn,
                                 preferred_element_type=jnp.float32)
    r_ref[...] = lax.dot_general(xb, wr_ref[...], dn,
                                 preferred_element_type=jnp.float32)


def _mid_body(a0_ref, a1_ref, d0_ref, d1_ref, r1_ref, b1_ref,
              w2l_ref, w2r_ref, b2_ref, p2_ref, r2b_ref):
    agg = a0_ref[...] + a1_ref[...]
    deg = jnp.maximum(d0_ref[...] + d1_ref[...], 1.0)
    h = jnp.maximum(agg / deg + b1_ref[...] + r1_ref[...], 0.0)
    dn = (((1,), (1,)), ((), ()))
    p2_ref[...] = lax.dot_general(h, w2l_ref[...], dn,
                                  preferred_element_type=jnp.float32)
    r2b_ref[...] = lax.dot_general(h, w2r_ref[...], dn,
                                   preferred_element_type=jnp.float32) + b2_ref[...]


def _final_body(a0_ref, a1_ref, d0_ref, d1_ref, r2b_ref, batch_ref,
                out_ref, sum_acc, cnt_acc):
    i = pl.program_id(0)

    @pl.when(i == 0)
    def _():
        sum_acc[...] = jnp.zeros_like(sum_acc)
        cnt_acc[...] = jnp.zeros_like(cnt_acc)

    agg = a0_ref[...] + a1_ref[...]
    deg = jnp.maximum(d0_ref[...] + d1_ref[...], 1.0)
    node = agg / deg + r2b_ref[...]

    bvec = batch_ref[0]
    ohT = (lax.broadcasted_iota(jnp.int32, (G, BN), 0)
           == jnp.broadcast_to(bvec, (G, BN))).astype(jnp.float32)
    dn = (((1,), (0,)), ((), ()))
    sum_acc[...] += lax.dot_general(ohT, node, dn,
                                    preferred_element_type=jnp.float32)
    cnt_acc[...] += jnp.sum(ohT, axis=1, keepdims=True)

    @pl.when(i == pl.num_programs(0) - 1)
    def _():
        out_ref[...] = sum_acc[...] / jnp.maximum(cnt_acc[...], 1.0)


def _row_spec(w):
    return pl.BlockSpec((BN, w), lambda i: (i, 0))


def _full_spec(shape):
    return pl.BlockSpec(shape, lambda i: tuple(0 for _ in shape))


def _proj1(x, W1_l, W1_r):
    return pl.pallas_call(
        _proj1_body,
        grid=(NB,),
        in_specs=[_row_spec(D), _full_spec((H, D)), _full_spec((H, D))],
        out_specs=[_row_spec(H), _row_spec(H)],
        out_shape=[jax.ShapeDtypeStruct((N, H), jnp.float32)] * 2,
    )(x, W1_l, W1_r)


def _mid(a0, a1, d0, d1, R1, b1, W2_l, W2_r, b2):
    return pl.pallas_call(
        _mid_body,
        grid=(NB,),
        in_specs=[_row_spec(16), _row_spec(16), _row_spec(16), _row_spec(16),
                  _row_spec(H), _full_spec((1, H)),
                  _full_spec((O, H)), _full_spec((O, H)), _full_spec((1, O))],
        out_specs=[_row_spec(O), _row_spec(O)],
        out_shape=[jax.ShapeDtypeStruct((N, O), jnp.float32)] * 2,
    )(a0, a1, d0, d1, R1, b1, W2_l, W2_r, b2)


def _final(a0, a1, d0, d1, R2b, batch3):
    return pl.pallas_call(
        _final_body,
        grid=(NB,),
        in_specs=[_row_spec(16), _row_spec(16), _row_spec(16), _row_spec(16),
                  _row_spec(O),
                  pl.BlockSpec((1, 1, BN), lambda i: (i, 0, 0))],
        out_specs=pl.BlockSpec((G, O), lambda i: (0, 0)),
        out_shape=jax.ShapeDtypeStruct((G, O), jnp.float32),
        scratch_shapes=[pltpu.VMEM((G, O), jnp.float32),
                        pltpu.VMEM((G, 1), jnp.float32)],
    )(a0, a1, d0, d1, R2b, batch3)


_sc_pass1 = _make_sc_scatter(True)
_sc_pass2 = _make_sc_scatter(False)


def kernel(x, edge_index, batch, W1_l, b1_l, W1_r, W2_l, b2_l, W2_r):
    src = edge_index[0].astype(jnp.int32)
    dst = edge_index[1].astype(jnp.int32)
    b1 = b1_l.reshape(1, H).astype(jnp.float32)
    b2 = b2_l.reshape(1, O).astype(jnp.float32)
    batch3 = batch.astype(jnp.int32).reshape(NB, 1, BN)
    x = x.astype(jnp.float32)

    P1, R1 = _proj1(x, W1_l, W1_r)
    a0, a1, d0, d1 = _sc_pass1(P1, src, dst)
    P2, R2b = _mid(a0, a1, d0, d1, R1, b1, W2_l, W2_r, b2)
    g0, g1 = _sc_pass2(P2, src, dst)
    return _final(g0, g1, d0, d1, R2b, batch3)


# speedup vs baseline: 37.0409x; 1.4596x over previous
import functools

import jax
import jax.numpy as jnp
from jax import lax
from jax.experimental import pallas as pl
from jax.experimental.pallas import tpu as pltpu
from jax.experimental.pallas import tpu_sc as plsc

N = 10000
E = 320000
D = 128
H = 16
O = 16
G = 128

NC = 2
NS = 16
NW = NC * NS
EP = E // NW
CH = 1000
NCH = EP // CH
NZ = N // NS
NP = N // 8


def _sc_scatter_body(with_deg, *refs):
    if with_deg:
        (p_hbm, ei_hbm, agg0, agg1, deg0, deg1,
         idx_all, dst_all, rows0, rows1, ones_v, agg_sh, deg_sh,
         isem, gsem, asem, dsem) = refs
    else:
        (p_hbm, ei_hbm, agg0, agg1,
         idx_all, dst_all, rows0, rows1, agg_sh,
         isem, gsem, asem) = refs

    c = lax.axis_index("c")
    s = lax.axis_index("s")
    wid = s * NC + c
    base = wid * EP

    idesc = []
    for i in range(NCH):
        off = base + i * CH
        idesc.append(pltpu.async_copy(
            ei_hbm.at[0, pl.ds(off, CH)], idx_all.at[i], isem))
        idesc.append(pltpu.async_copy(
            ei_hbm.at[1, pl.ds(off, CH)], dst_all.at[i], isem))

    def _zrow(i, _):
        rows0[i, :] = jnp.zeros((16,), jnp.float32)
        return 0
    lax.fori_loop(0, NZ, _zrow, 0)
    if with_deg:
        def _orow(i, _):
            ones_v[i, :] = jnp.ones((16,), jnp.float32)
            return 0
        lax.fori_loop(0, CH, _orow, 0)

    pltpu.sync_copy(rows0.at[pl.ds(0, NZ)], agg_sh.at[pl.ds(s * NZ, NZ)])
    if with_deg:
        pltpu.sync_copy(rows0.at[pl.ds(0, NZ)], deg_sh.at[pl.ds(s * NZ, NZ)])
    plsc.subcore_barrier()
    for d in idesc:
        d.wait()

    rows = (rows0, rows1)
    g = [None] * NCH
    a = [None] * NCH
    dd = []
    g[0] = pltpu.async_copy(p_hbm.at[idx_all.at[0]], rows[0], gsem)
    for i in range(NCH):
        if i + 1 < NCH:
            if i >= 1:
                a[i - 1].wait()
            g[i + 1] = pltpu.async_copy(
                p_hbm.at[idx_all.at[i + 1]], rows[(i + 1) % 2], gsem)
        g[i].wait()
        a[i] = pltpu.async_copy(
            rows[i % 2], agg_sh.at[dst_all.at[i]], asem, add=True)
        if with_deg:
            dd.append(pltpu.async_copy(
                ones_v, deg_sh.at[dst_all.at[i]], dsem, add=True))
    for i in range(max(0, NCH - 2), NCH):
        a[i].wait()
    for d in dd:
        d.wait()
    plsc.subcore_barrier()

    rb = s * NZ

    @pl.when(c == 0)
    def _():
        pltpu.sync_copy(agg_sh.at[pl.ds(rb, NZ)], agg0.at[pl.ds(rb, NZ)])
        if with_deg:
            pltpu.sync_copy(deg_sh.at[pl.ds(rb, NZ)], deg0.at[pl.ds(rb, NZ)])

    @pl.when(c == 1)
    def _():
        pltpu.sync_copy(agg_sh.at[pl.ds(rb, NZ)], agg1.at[pl.ds(rb, NZ)])
        if with_deg:
            pltpu.sync_copy(deg_sh.at[pl.ds(rb, NZ)], deg1.at[pl.ds(rb, NZ)])


def _make_sc_scatter(with_deg):
    n_out = 4 if with_deg else 2
    scratch = [
        pltpu.VMEM((NCH, CH), jnp.int32),
        pltpu.VMEM((NCH, CH), jnp.int32),
        pltpu.VMEM((CH, 16), jnp.float32),
        pltpu.VMEM((CH, 16), jnp.float32),
    ]
    if with_deg:
        scratch.append(pltpu.VMEM((CH, 16), jnp.float32))
    scratch.append(pltpu.VMEM_SHARED((N, 16), jnp.float32))
    if with_deg:
        scratch.append(pltpu.VMEM_SHARED((N, 16), jnp.float32))
    scratch.extend([pltpu.SemaphoreType.DMA] * (4 if with_deg else 3))

    mesh = plsc.VectorSubcoreMesh(core_axis_name="c", subcore_axis_name="s")
    return functools.partial(
        pl.kernel,
        mesh=mesh,
        out_type=[jax.ShapeDtypeStruct((N, 16), jnp.float32)] * n_out,
        scratch_types=scratch,
        compiler_params=pltpu.CompilerParams(use_tc_tiling_on_sc=False),
    )(functools.partial(_sc_scatter_body, with_deg))


def _full_spec(shape):
    return pl.BlockSpec(shape, lambda: tuple(0 for _ in shape))


def _dot(a, b):
    return lax.dot_general(a, b, (((1,), (0,)), ((), ())),
                           preferred_element_type=jnp.float32)


def _projp_body(xp_ref, wl_ref, wr_ref, p_ref, r_ref):
    xp = xp_ref[...]
    p_ref[...] = _dot(xp, wl_ref[...])
    r_ref[...] = _dot(xp, wr_ref[...])


def _midp_body(a0_ref, a1_ref, d0_ref, d1_ref, r1_ref, b1_ref,
               w2l_ref, w2r_ref, b2_ref, p2_ref, r2b_ref):
    deg = jnp.maximum(d0_ref[...] + d1_ref[...], 1.0)
    h = jnp.maximum((a0_ref[...] + a1_ref[...]) / deg
                    + b1_ref[...] + r1_ref[...], 0.0)
    p2_ref[...] = _dot(h, w2l_ref[...])
    r2b_ref[...] = _dot(h, w2r_ref[...]) + b2_ref[...]


def _finalp_body(g0_ref, g1_ref, d0_ref, d1_ref, r2b_ref, bt_ref, out_ref):
    deg = jnp.maximum(d0_ref[...] + d1_ref[...], 1.0)
    nodep = (g0_ref[...] + g1_ref[...]) / deg + r2b_ref[...]
    acc = jnp.zeros((G, O), jnp.float32)
    cnt = jnp.zeros((G, 1), jnp.float32)
    for i in range(8):
        bv = bt_ref[i:i + 1, :]
        oh = (lax.broadcasted_iota(jnp.int32, (G, NP), 0)
              == jnp.broadcast_to(bv, (G, NP))).astype(jnp.float32)
        acc += _dot(oh, nodep[:, 16 * i:16 * i + 16])
        cnt += jnp.sum(oh, axis=1, keepdims=True)
    out_ref[...] = acc / jnp.maximum(cnt, 1.0)


def _projp(xp, bigwl, bigwr):
    return pl.pallas_call(
        _projp_body,
        in_specs=[_full_spec((NP, 8 * D)), _full_spec((8 * D, 128)),
                  _full_spec((8 * D, 128))],
        out_specs=[_full_spec((NP, 128))] * 2,
        out_shape=[jax.ShapeDtypeStruct((NP, 128), jnp.float32)] * 2,
    )(xp, bigwl, bigwr)


def _midp(a0p, a1p, d0p, d1p, r1p, b1p, w2lbd, w2rbd, b2p):
    return pl.pallas_call(
        _midp_body,
        in_specs=[_full_spec((NP, 128))] * 5
        + [_full_spec((1, 128)), _full_spec((128, 128)),
           _full_spec((128, 128)), _full_spec((1, 128))],
        out_specs=[_full_spec((NP, 128))] * 2,
        out_shape=[jax.ShapeDtypeStruct((NP, 128), jnp.float32)] * 2,
    )(a0p, a1p, d0p, d1p, r1p, b1p, w2lbd, w2rbd, b2p)


def _finalp(g0p, g1p, d0p, d1p, r2bp, bt):
    return pl.pallas_call(
        _finalp_body,
        in_specs=[_full_spec((NP, 128))] * 5 + [_full_spec((8, NP))],
        out_specs=_full_spec((G, O)),
        out_shape=jax.ShapeDtypeStruct((G, O), jnp.float32),
    )(g0p, g1p, d0p, d1p, r2bp, bt)


_sc_pass1 = _make_sc_scatter(True)
_sc_pass2 = _make_sc_scatter(False)


def kernel(x, edge_index, batch, W1_l, b1_l, W1_r, W2_l, b2_l, W2_r):
    f32 = jnp.float32
    ei = edge_index.astype(jnp.int32)
    xp = x.astype(f32).reshape(NP, 8 * D)
    e8 = jnp.eye(8, dtype=f32)
    bigwl = jnp.kron(e8, W1_l.T.astype(f32))
    bigwr = jnp.kron(e8, W1_r.T.astype(f32))
    w2lbd = jnp.kron(e8, W2_l.T.astype(f32))
    w2rbd = jnp.kron(e8, W2_r.T.astype(f32))
    b1p = jnp.tile(b1_l.astype(f32).reshape(1, H), (1, 8))
    b2p = jnp.tile(b2_l.astype(f32).reshape(1, O), (1, 8))
    bt = batch.astype(jnp.int32).reshape(NP, 8).T

    P1p, R1p = _projp(xp, bigwl, bigwr)
    a0, a1, d0, d1 = _sc_pass1(P1p.reshape(N, H), ei)
    P2p, R2bp = _midp(a0.reshape(NP, 128), a1.reshape(NP, 128),
                      d0.reshape(NP, 128), d1.reshape(NP, 128),
                      R1p, b1p, w2lbd, w2rbd, b2p)
    g0, g1 = _sc_pass2(P2p.reshape(N, O), ei)
    return _finalp(g0.reshape(NP, 128), g1.reshape(NP, 128),
                   d0.reshape(NP, 128), d1.reshape(NP, 128), R2bp, bt)


# speedup vs baseline: 38.1181x; 1.0291x over previous
import functools

import jax
import jax.numpy as jnp
from jax import lax
from jax.experimental import pallas as pl
from jax.experimental.pallas import tpu as pltpu
from jax.experimental.pallas import tpu_sc as plsc

N = 10000
E = 320000
D = 128
H = 16
O = 16
G = 128

NC = 2
NS = 16
NW = NC * NS
EP = E // NW
CH = 1000
NCH = EP // CH
NZ = N // NS
NP = N // 8


def _sc_scatter_body(with_deg, *refs):
    if with_deg:
        (p_hbm, ei_hbm, agg0, agg1, deg0, deg1,
         idx_all, dst_all, rows0, rows1, ones_v, agg_sh, deg_sh,
         isem, gsem, asem, dsem) = refs
    else:
        (p_hbm, ei_hbm, agg0, agg1,
         idx_all, dst_all, rows0, rows1, agg_sh,
         isem, gsem, asem) = refs

    c = lax.axis_index("c")
    s = lax.axis_index("s")
    wid = s * NC + c
    base = wid * EP

    idesc = []
    for i in range(NCH):
        off = base + i * CH
        idesc.append(pltpu.async_copy(
            ei_hbm.at[0, pl.ds(off, CH)], idx_all.at[i], isem))
        idesc.append(pltpu.async_copy(
            ei_hbm.at[1, pl.ds(off, CH)], dst_all.at[i], isem))

    def _zrow(i, _):
        rows0[i, :] = jnp.zeros((16,), jnp.float32)
        return 0
    lax.fori_loop(0, NZ, _zrow, 0)
    if with_deg:
        def _orow(i, _):
            ones_v[i, :] = jnp.ones((16,), jnp.float32)
            return 0
        lax.fori_loop(0, CH, _orow, 0)

    pltpu.sync_copy(rows0.at[pl.ds(0, NZ)], agg_sh.at[pl.ds(s * NZ, NZ)])
    if with_deg:
        pltpu.sync_copy(rows0.at[pl.ds(0, NZ)], deg_sh.at[pl.ds(s * NZ, NZ)])
    plsc.subcore_barrier()
    for d in idesc:
        d.wait()

    rows = (rows0, rows1)
    g = [None] * NCH
    a = [None] * NCH
    dd = []
    g[0] = pltpu.async_copy(p_hbm.at[idx_all.at[0]], rows[0], gsem)
    for i in range(NCH):
        if i + 1 < NCH:
            if i >= 1:
                a[i - 1].wait()
            g[i + 1] = pltpu.async_copy(
                p_hbm.at[idx_all.at[i + 1]], rows[(i + 1) % 2], gsem)
        g[i].wait()
        a[i] = pltpu.async_copy(
            rows[i % 2], agg_sh.at[dst_all.at[i]], asem, add=True)
        if with_deg:
            dd.append(pltpu.async_copy(
                ones_v, deg_sh.at[dst_all.at[i]], dsem, add=True))
    for i in range(max(0, NCH - 2), NCH):
        a[i].wait()
    for d in dd:
        d.wait()
    plsc.subcore_barrier()

    rb = s * NZ

    @pl.when(c == 0)
    def _():
        pltpu.sync_copy(agg_sh.at[pl.ds(rb, NZ)], agg0.at[pl.ds(rb, NZ)])
        if with_deg:
            pltpu.sync_copy(deg_sh.at[pl.ds(rb, NZ)], deg0.at[pl.ds(rb, NZ)])

    @pl.when(c == 1)
    def _():
        pltpu.sync_copy(agg_sh.at[pl.ds(rb, NZ)], agg1.at[pl.ds(rb, NZ)])
        if with_deg:
            pltpu.sync_copy(deg_sh.at[pl.ds(rb, NZ)], deg1.at[pl.ds(rb, NZ)])


def _make_sc_scatter(with_deg):
    n_out = 4 if with_deg else 2
    scratch = [
        pltpu.VMEM((NCH, CH), jnp.int32),
        pltpu.VMEM((NCH, CH), jnp.int32),
        pltpu.VMEM((CH, 16), jnp.float32),
        pltpu.VMEM((CH, 16), jnp.float32),
    ]
    if with_deg:
        scratch.append(pltpu.VMEM((CH, 16), jnp.float32))
    scratch.append(pltpu.VMEM_SHARED((N, 16), jnp.float32))
    if with_deg:
        scratch.append(pltpu.VMEM_SHARED((N, 16), jnp.float32))
    scratch.extend([pltpu.SemaphoreType.DMA] * (4 if with_deg else 3))

    mesh = plsc.VectorSubcoreMesh(core_axis_name="c", subcore_axis_name="s")
    return functools.partial(
        pl.kernel,
        mesh=mesh,
        out_type=[jax.ShapeDtypeStruct((N, 16), jnp.float32)] * n_out,
        scratch_types=scratch,
        compiler_params=pltpu.CompilerParams(use_tc_tiling_on_sc=False),
    )(functools.partial(_sc_scatter_body, with_deg))


def _full_spec(shape):
    return pl.BlockSpec(shape, lambda: tuple(0 for _ in shape))


def _dot(a, b):
    return lax.dot_general(a, b, (((1,), (0,)), ((), ())),
                           preferred_element_type=jnp.float32)


def _dot_t(a, b):
    return lax.dot_general(a, b, (((1,), (1,)), ((), ())),
                           preferred_element_type=jnp.float32)


def _blockdiag_matmul(xin, w_t, din):
    outs = [_dot_t(xin[:, din * i:din * (i + 1)], w_t) for i in range(8)]
    return jnp.concatenate(outs, axis=1)


def _projp_body(xp_ref, wl_ref, wr_ref, p_ref, r_ref):
    xp = xp_ref[...]
    p_ref[...] = _blockdiag_matmul(xp, wl_ref[...], D)
    r_ref[...] = _blockdiag_matmul(xp, wr_ref[...], D)


def _midp_body(a0_ref, a1_ref, d0_ref, d1_ref, r1_ref, b1_ref,
               w2l_ref, w2r_ref, b2_ref, p2_ref, r2b_ref):
    deg = jnp.maximum(d0_ref[...] + d1_ref[...], 1.0)
    h = jnp.maximum((a0_ref[...] + a1_ref[...]) / deg
                    + b1_ref[...] + r1_ref[...], 0.0)
    p2_ref[...] = _blockdiag_matmul(h, w2l_ref[...], H)
    r2b_ref[...] = _blockdiag_matmul(h, w2r_ref[...], H) + b2_ref[...]


def _finalp_body(g0_ref, g1_ref, d0_ref, d1_ref, r2b_ref, bt_ref, out_ref):
    deg = jnp.maximum(d0_ref[...] + d1_ref[...], 1.0)
    nodep = (g0_ref[...] + g1_ref[...]) / deg + r2b_ref[...]
    acc = jnp.zeros((G, O), jnp.float32)
    cnt = jnp.zeros((G, 1), jnp.float32)
    for i in range(8):
        bv = bt_ref[i:i + 1, :]
        oh = (lax.broadcasted_iota(jnp.int32, (G, NP), 0)
              == jnp.broadcast_to(bv, (G, NP))).astype(jnp.float32)
        acc += _dot(oh, nodep[:, 16 * i:16 * i + 16])
        cnt += jnp.sum(oh, axis=1, keepdims=True)
    out_ref[...] = acc / jnp.maximum(cnt, 1.0)


def _projp(xp, wl, wr):
    return pl.pallas_call(
        _projp_body,
        in_specs=[_full_spec((NP, 8 * D)), _full_spec((H, D)),
                  _full_spec((H, D))],
        out_specs=[_full_spec((NP, 128))] * 2,
        out_shape=[jax.ShapeDtypeStruct((NP, 128), jnp.float32)] * 2,
    )(xp, wl, wr)


def _midp(a0p, a1p, d0p, d1p, r1p, b1p, w2lbd, w2rbd, b2p):
    return pl.pallas_call(
        _midp_body,
        in_specs=[_full_spec((NP, 128))] * 5
        + [_full_spec((1, 128)), _full_spec((O, H)),
           _full_spec((O, H)), _full_spec((1, 128))],
        out_specs=[_full_spec((NP, 128))] * 2,
        out_shape=[jax.ShapeDtypeStruct((NP, 128), jnp.float32)] * 2,
    )(a0p, a1p, d0p, d1p, r1p, b1p, w2lbd, w2rbd, b2p)


def _finalp(g0p, g1p, d0p, d1p, r2bp, bt):
    return pl.pallas_call(
        _finalp_body,
        in_specs=[_full_spec((NP, 128))] * 5 + [_full_spec((8, NP))],
        out_specs=_full_spec((G, O)),
        out_shape=jax.ShapeDtypeStruct((G, O), jnp.float32),
    )(g0p, g1p, d0p, d1p, r2bp, bt)


_sc_pass1 = _make_sc_scatter(True)
_sc_pass2 = _make_sc_scatter(False)


def kernel(x, edge_index, batch, W1_l, b1_l, W1_r, W2_l, b2_l, W2_r):
    f32 = jnp.float32
    ei = edge_index.astype(jnp.int32)
    xp = x.astype(f32).reshape(NP, 8 * D)
    b1p = jnp.tile(b1_l.astype(f32).reshape(1, H), (1, 8))
    b2p = jnp.tile(b2_l.astype(f32).reshape(1, O), (1, 8))
    bt = batch.astype(jnp.int32).reshape(NP, 8).T

    P1p, R1p = _projp(xp, W1_l.astype(f32), W1_r.astype(f32))
    a0, a1, d0, d1 = _sc_pass1(P1p.reshape(N, H), ei)
    P2p, R2bp = _midp(a0.reshape(NP, 128), a1.reshape(NP, 128),
                      d0.reshape(NP, 128), d1.reshape(NP, 128),
                      R1p, b1p, W2_l.astype(f32), W2_r.astype(f32), b2p)
    g0, g1 = _sc_pass2(P2p.reshape(N, O), ei)
    return _finalp(g0.reshape(NP, 128), g1.reshape(NP, 128),
                   d0.reshape(NP, 128), d1.reshape(NP, 128), R2bp, bt)
